# baseline (speedup 1.0000x reference)
import functools

import jax
import jax.numpy as jnp
from jax import lax
from jax.experimental import pallas as pl
from jax.experimental.pallas import tpu as pltpu
from jax.experimental.pallas import tpu_sc as plsc

B = 2
N_NODES = 10000
N_EDGES = 320000
D_NODE = 128
D_EDGE = 16
D_GLOBAL = 32
EDGE_OUT = 16
NODE_OUT = 128
GLOBAL_OUT = 32

NUM_TILES = 16
ET = 128
N_ETILES = N_EDGES // ET
ET_PER_TILE = 156
K_SUB = 6
CHUNK = K_SUB * ET
N_CHUNKS = ET_PER_TILE // K_SUB
REM_ETILES = N_ETILES - NUM_TILES * ET_PER_TILE
AGG_COPY_TILES = 10
AGG_COPY_ROWS = N_NODES // AGG_COPY_TILES


def _tables_body(nodes_ref, wr_ref, ws_ref, tabr_ref, tabs_ref):
    n = nodes_ref[0]
    tabr_ref[0] = jnp.dot(n, wr_ref[...], preferred_element_type=jnp.float32)
    tabs_ref[0] = jnp.dot(n, ws_ref[...], preferred_element_type=jnp.float32)


def _node_tables(nodes, we_r, we_s):
    return pl.pallas_call(
        _tables_body,
        grid=(B,),
        in_specs=[
            pl.BlockSpec((1, N_NODES, D_NODE), lambda b: (b, 0, 0)),
            pl.BlockSpec((D_NODE, EDGE_OUT), lambda b: (0, 0)),
            pl.BlockSpec((D_NODE, EDGE_OUT), lambda b: (0, 0)),
        ],
        out_specs=[
            pl.BlockSpec((1, N_NODES, EDGE_OUT), lambda b: (b, 0, 0)),
            pl.BlockSpec((1, N_NODES, EDGE_OUT), lambda b: (b, 0, 0)),
        ],
        out_shape=[
            jax.ShapeDtypeStruct((B, N_NODES, EDGE_OUT), jnp.float32),
            jax.ShapeDtypeStruct((B, N_NODES, EDGE_OUT), jnp.float32),
        ],
    )(nodes, we_r, we_s)


EDGE_BLK = 16000
N_EDGE_BLKS = N_EDGES // EDGE_BLK


def _edge_base_body(edges_t_ref, we_e_t_ref, g_ref, we_g_t_ref, be_ref, out_ref):
    base = jnp.dot(we_e_t_ref[...], edges_t_ref[0],
                   preferred_element_type=jnp.float32)
    gterm = jnp.dot(we_g_t_ref[...], g_ref[0],
                    preferred_element_type=jnp.float32)
    out_ref[0] = base + gterm + be_ref[...]


def _edge_base(edges_t, we_e_t, gcol, we_g_t, be_col):
    return pl.pallas_call(
        _edge_base_body,
        grid=(B, N_EDGE_BLKS),
        in_specs=[
            pl.BlockSpec((1, D_EDGE, EDGE_BLK), lambda b, i: (b, 0, i)),
            pl.BlockSpec((EDGE_OUT, D_EDGE), lambda b, i: (0, 0)),
            pl.BlockSpec((1, D_GLOBAL, 1), lambda b, i: (b, 0, 0)),
            pl.BlockSpec((EDGE_OUT, D_GLOBAL), lambda b, i: (0, 0)),
            pl.BlockSpec((EDGE_OUT, 1), lambda b, i: (0, 0)),
        ],
        out_specs=pl.BlockSpec((1, EDGE_OUT, EDGE_BLK), lambda b, i: (b, 0, i)),
        out_shape=jax.ShapeDtypeStruct((B, EDGE_OUT, N_EDGES), jnp.float32),
    )(edges_t, we_e_t, gcol, we_g_t, be_col)


def _sc_edges(base_pk, tabr_flat, tabs_flat, grecv2, gsend2, lrecv2):
    mesh = plsc.VectorSubcoreMesh(core_axis_name="c", subcore_axis_name="s")

    @functools.partial(
        pl.kernel,
        out_type=[
            jax.ShapeDtypeStruct((B, 2, N_ETILES, 8, ET), jnp.float32),
            jax.ShapeDtypeStruct((B * N_NODES, EDGE_OUT), jnp.float32),
        ],
        mesh=mesh,
        scratch_types=[
            pltpu.VMEM((3, K_SUB, ET), jnp.int32),
            pltpu.VMEM((3, K_SUB, ET), jnp.int32),
            pltpu.VMEM((3, K_SUB, ET), jnp.int32),
            pltpu.VMEM((3, 2, K_SUB, 8, ET), jnp.float32),
            pltpu.VMEM((2, CHUNK, EDGE_OUT), jnp.float32),
            pltpu.VMEM((2, CHUNK, EDGE_OUT), jnp.float32),
            pltpu.VMEM_SHARED((N_NODES, EDGE_OUT), jnp.float32),
            pltpu.SemaphoreType.DMA,
            pltpu.SemaphoreType.DMA,
            pltpu.SemaphoreType.DMA,
            pltpu.SemaphoreType.DMA,
            pltpu.SemaphoreType.DMA,
            pltpu.SemaphoreType.DMA,
        ],
        compiler_params=pltpu.CompilerParams(use_tc_tiling_on_sc=False,
                                             needs_layout_passes=False),
    )
    def sc_kernel(base_hbm, tabr_hbm, tabs_hbm, grecv_hbm, gsend_hbm,
                  lrecv_hbm, eout_hbm, agg_hbm, idxr_v, idxs_v, idxl_v,
                  bh, rbuf, sbuf, agg_sh, sem_pf0, sem_pf1, sem_pf2,
                  sem_g, sem_o, sem_sc):
        c = lax.axis_index("c")
        s = lax.axis_index("s")
        sem_pf = [sem_pf0, sem_pf1, sem_pf2]

        npt = N_NODES // NUM_TILES

        def _zero(j, _):
            rbuf[0, j, :] = jnp.zeros((EDGE_OUT,), jnp.float32)
            return 0
        lax.fori_loop(0, npt, _zero, 0)
        pltpu.sync_copy(rbuf.at[0, pl.ds(0, npt)],
                        agg_sh.at[pl.ds(s * npt, npt)])
        plsc.subcore_barrier()

        iota = lax.iota(jnp.int32, 16)
        hvec = lax.shift_right_logical(iota, 3)
        rvec = lax.bitwise_and(iota, 7)

        def et0_of(k):
            return pl.multiple_of(s * ET_PER_TILE + k * K_SUB, K_SUB)

        def fire_pf(k):
            p = k % 3
            et0 = et0_of(k)
            row0 = c * N_ETILES + et0
            ds = [
                pltpu.async_copy(grecv_hbm.at[pl.ds(row0, K_SUB)],
                                 idxr_v.at[p], sem_pf[p]),
                pltpu.async_copy(gsend_hbm.at[pl.ds(row0, K_SUB)],
                                 idxs_v.at[p], sem_pf[p]),
                pltpu.async_copy(lrecv_hbm.at[pl.ds(row0, K_SUB)],
                                 idxl_v.at[p], sem_pf[p]),
            ]
            for h in range(2):
                ds.append(pltpu.async_copy(base_hbm.at[c, h, pl.ds(et0, K_SUB)],
                                           bh.at[p, h], sem_pf[p]))
            return ds

        def fire_g(k):
            p, q = k % 3, k % 2
            ds = []
            for j in range(K_SUB):
                ds.append(pltpu.async_copy(tabr_hbm.at[idxr_v.at[p, j]],
                                           rbuf.at[q, pl.ds(j * ET, ET)],
                                           sem_g))
                ds.append(pltpu.async_copy(tabs_hbm.at[idxs_v.at[p, j]],
                                           sbuf.at[q, pl.ds(j * ET, ET)],
                                           sem_g))
            return ds

        def compute(k):
            p, q = k % 3, k % 2
            bh_p = bh.at[p]

            def et_body(t, _):
                tvec = jnp.full((16,), t, jnp.int32)
                j0 = t * ET

                @plsc.parallel_loop(0, ET, unroll=16)
                def _(l):
                    lvec = jnp.full((16,), l, jnp.int32)
                    base_v = plsc.load_gather(bh_p, [hvec, tvec, rvec, lvec])
                    j = j0 + l
                    v = jnp.maximum(base_v + rbuf[q, j, :] + sbuf[q, j, :], 0.0)
                    plsc.store_scatter(bh_p, [hvec, tvec, rvec, lvec], v)
                    rbuf[q, j, :] = v
                return 0
            lax.fori_loop(0, K_SUB, et_body, 0)

        def fire_o(k):
            p, q = k % 3, k % 2
            et0 = et0_of(k)
            ds = []
            for h in range(2):
                ds.append(pltpu.async_copy(bh.at[p, h],
                                           eout_hbm.at[c, h, pl.ds(et0, K_SUB)],
                                           sem_o))
            for j in range(K_SUB):
                ds.append(pltpu.async_copy(rbuf.at[q, pl.ds(j * ET, ET)],
                                           agg_sh.at[idxl_v.at[p, j]],
                                           sem_sc, add=True))
            return ds

        def drain(ds):
            for d in ds:
                d.wait()

        pf_d = {0: fire_pf(0), 1: fire_pf(1)}
        drain(pf_d.pop(0))
        g_d = {0: fire_g(0)}
        o_prev = None
        for k in range(N_CHUNKS):
            drain(g_d.pop(k))
            compute(k)
            if o_prev is not None:
                drain(o_prev)
            o_prev = fire_o(k)
            if k + 2 < N_CHUNKS:
                pf_d[k + 2] = fire_pf(k + 2)
            if k + 1 < N_CHUNKS:
                drain(pf_d.pop(k + 1))
                g_d[k + 1] = fire_g(k + 1)
        drain(o_prev)

        @pl.when(s < REM_ETILES)
        def _():
            et0 = NUM_TILES * ET_PER_TILE + s
            row0 = c * N_ETILES + et0
            pltpu.sync_copy(grecv_hbm.at[pl.ds(row0, 1)],
                            idxr_v.at[0, pl.ds(0, 1)])
            pltpu.sync_copy(gsend_hbm.at[pl.ds(row0, 1)],
                            idxs_v.at[0, pl.ds(0, 1)])
            pltpu.sync_copy(lrecv_hbm.at[pl.ds(row0, 1)],
                            idxl_v.at[0, pl.ds(0, 1)])
            for h in range(2):
                pltpu.sync_copy(base_hbm.at[c, h, pl.ds(et0, 1)],
                                bh.at[0, h, pl.ds(0, 1)])
            drain([pltpu.async_copy(tabr_hbm.at[idxr_v.at[0, 0]],
                                    rbuf.at[0, pl.ds(0, ET)], sem_g),
                   pltpu.async_copy(tabs_hbm.at[idxs_v.at[0, 0]],
                                    sbuf.at[0, pl.ds(0, ET)], sem_g)])
            bh_p = bh.at[0]

            def row_body(j, _):
                lvec = jnp.full((16,), j, jnp.int32)
                tvec = jnp.zeros((16,), jnp.int32)
                base_v = plsc.load_gather(bh_p, [hvec, tvec, rvec, lvec])
                v = jnp.maximum(base_v + rbuf[0, j, :] + sbuf[0, j, :], 0.0)
                plsc.store_scatter(bh_p, [hvec, tvec, rvec, lvec], v)
                rbuf[0, j, :] = v
                return 0
            lax.fori_loop(0, ET, row_body, 0)
            for h in range(2):
                pltpu.sync_copy(bh.at[0, h, pl.ds(0, 1)],
                                eout_hbm.at[c, h, pl.ds(et0, 1)])
            pltpu.sync_copy(rbuf.at[0, pl.ds(0, ET)],
                            agg_sh.at[idxl_v.at[0, 0]], add=True)

        plsc.subcore_barrier()
        n0 = s * npt
        pltpu.sync_copy(agg_sh.at[pl.ds(n0, npt)],
                        agg_hbm.at[pl.ds(c * N_NODES + n0, npt)])

    return sc_kernel(base_pk, tabr_flat, tabs_flat, grecv2, gsend2, lrecv2)


NODE_BLK = 1000
N_NODE_BLKS = N_NODES // NODE_BLK


def _node_global_body(agg_ref, nodes_ref, g_ref, wn_a_ref, wn_n_ref, wn_g_ref,
                      bn_ref, wg_e_ref, wg_n_ref, wg_g_ref, bg_ref,
                      nout_ref, gout_ref, nsum_ref, esum_ref):
    b = pl.program_id(0)
    i = pl.program_id(1)
    agg = agg_ref[0]
    x = (jnp.dot(agg, wn_a_ref[...], preferred_element_type=jnp.float32)
         + jnp.dot(nodes_ref[0], wn_n_ref[...], preferred_element_type=jnp.float32)
         + jnp.dot(g_ref[0], wn_g_ref[...], preferred_element_type=jnp.float32)
         + bn_ref[...])
    n_out = jnp.maximum(x, 0.0)
    nout_ref[0] = n_out
    ns = jnp.sum(n_out, axis=0, keepdims=True)
    es = jnp.sum(agg, axis=0, keepdims=True)

    @pl.when(i == 0)
    def _():
        nsum_ref[...] = ns
        esum_ref[...] = es

    @pl.when(i > 0)
    def _():
        nsum_ref[...] += ns
        esum_ref[...] += es

    @pl.when(i == N_NODE_BLKS - 1)
    def _():
        gi = (jnp.dot(esum_ref[...], wg_e_ref[...], preferred_element_type=jnp.float32)
              + jnp.dot(nsum_ref[...], wg_n_ref[...], preferred_element_type=jnp.float32)
              + jnp.dot(g_ref[0], wg_g_ref[...], preferred_element_type=jnp.float32)
              + bg_ref[...])
        gout_ref[pl.ds(b, 1), :] = jnp.maximum(gi, 0.0)


def _node_global(agg, nodes, g3, wn_a, wn_n, wn_g, bn2, wg_e, wg_n, wg_g, bg2):
    return pl.pallas_call(
        _node_global_body,
        grid=(B, N_NODE_BLKS),
        in_specs=[
            pl.BlockSpec((1, NODE_BLK, EDGE_OUT), lambda b, i: (b, i, 0)),
            pl.BlockSpec((1, NODE_BLK, D_NODE), lambda b, i: (b, i, 0)),
            pl.BlockSpec((1, 1, D_GLOBAL), lambda b, i: (b, 0, 0)),
            pl.BlockSpec((EDGE_OUT, NODE_OUT), lambda b, i: (0, 0)),
            pl.BlockSpec((D_NODE, NODE_OUT), lambda b, i: (0, 0)),
            pl.BlockSpec((D_GLOBAL, NODE_OUT), lambda b, i: (0, 0)),
            pl.BlockSpec((1, NODE_OUT), lambda b, i: (0, 0)),
            pl.BlockSpec((EDGE_OUT, GLOBAL_OUT), lambda b, i: (0, 0)),
            pl.BlockSpec((NODE_OUT, GLOBAL_OUT), lambda b, i: (0, 0)),
            pl.BlockSpec((D_GLOBAL, GLOBAL_OUT), lambda b, i: (0, 0)),
            pl.BlockSpec((1, GLOBAL_OUT), lambda b, i: (0, 0)),
        ],
        out_specs=[
            pl.BlockSpec((1, NODE_BLK, NODE_OUT), lambda b, i: (b, i, 0)),
            pl.BlockSpec((B, GLOBAL_OUT), lambda b, i: (0, 0)),
        ],
        out_shape=[
            jax.ShapeDtypeStruct((B, N_NODES, NODE_OUT), jnp.float32),
            jax.ShapeDtypeStruct((B, GLOBAL_OUT), jnp.float32),
        ],
        scratch_shapes=[
            pltpu.VMEM((1, NODE_OUT), jnp.float32),
            pltpu.VMEM((1, EDGE_OUT), jnp.float32),
        ],
    )(agg, nodes, g3, wn_a, wn_n, wn_g, bn2, wg_e, wg_n, wg_g, bg2)


def kernel(nodes, edges, senders, receivers, global_feats, We, be, Wn, bn, Wg, bg):
    we_e, we_r = We[0:D_EDGE], We[D_EDGE:D_EDGE + D_NODE]
    we_s = We[D_EDGE + D_NODE:D_EDGE + 2 * D_NODE]
    we_g = We[D_EDGE + 2 * D_NODE:]
    wn_a, wn_n, wn_g = Wn[0:EDGE_OUT], Wn[EDGE_OUT:EDGE_OUT + D_NODE], Wn[EDGE_OUT + D_NODE:]
    wg_e, wg_n, wg_g = Wg[0:EDGE_OUT], Wg[EDGE_OUT:EDGE_OUT + NODE_OUT], Wg[EDGE_OUT + NODE_OUT:]

    g3 = global_feats.reshape(B, 1, D_GLOBAL)
    gcol = global_feats.reshape(B, D_GLOBAL, 1)
    be_col = be.reshape(EDGE_OUT, 1)
    bn2 = bn.reshape(1, NODE_OUT)
    bg2 = bg.reshape(1, GLOBAL_OUT)

    tabr, tabs = _node_tables(nodes, we_r, we_s)
    base_t = _edge_base(jnp.swapaxes(edges, 1, 2), we_e.T, gcol, we_g.T, be_col)
    base_pk = base_t.reshape(B, 2, 8, N_ETILES, ET).transpose(0, 1, 3, 2, 4)

    b_off = (jnp.arange(B, dtype=jnp.int32) * N_NODES)[:, None]
    grecv2 = (receivers + b_off).reshape(B * N_EDGES // ET, ET)
    gsend2 = (senders + b_off).reshape(B * N_EDGES // ET, ET)
    lrecv2 = receivers.reshape(B * N_EDGES // ET, ET)

    eout_pk, agg_flat = _sc_edges(
        base_pk,
        tabr.reshape(B * N_NODES, EDGE_OUT),
        tabs.reshape(B * N_NODES, EDGE_OUT),
        grecv2, gsend2, lrecv2)

    agg = agg_flat.reshape(B, N_NODES, EDGE_OUT)
    e_out = eout_pk.transpose(0, 2, 4, 1, 3).reshape(B, N_EDGES, EDGE_OUT)

    n_out, g_out = _node_global(agg, nodes, g3, wn_a, wn_n, wn_g, bn2,
                                wg_e, wg_n, wg_g, bg2)
    return (n_out, e_out, g_out)


# speedup vs baseline: 1.4724x; 1.4724x over previous
import functools

import jax
import jax.numpy as jnp
from jax import lax
from jax.experimental import pallas as pl
from jax.experimental.pallas import tpu as pltpu
from jax.experimental.pallas import tpu_sc as plsc

B = 2
N_NODES = 10000
N_EDGES = 320000
D_NODE = 128
D_EDGE = 16
D_GLOBAL = 32
EDGE_OUT = 16
NODE_OUT = 128
GLOBAL_OUT = 32

NUM_TILES = 16
ET = 128
N_ETILES = N_EDGES // ET
ET_PER_TILE = 156
K_SUB = 6
CHUNK = K_SUB * ET
N_CHUNKS = ET_PER_TILE // K_SUB
REM_ETILES = N_ETILES - NUM_TILES * ET_PER_TILE
AGG_COPY_TILES = 10
AGG_COPY_ROWS = N_NODES // AGG_COPY_TILES


def _tables_body(nodes_ref, wr_ref, ws_ref, tabr_ref, tabs_ref):
    n = nodes_ref[0]
    tabr_ref[0] = jnp.dot(n, wr_ref[...], preferred_element_type=jnp.float32)
    tabs_ref[0] = jnp.dot(n, ws_ref[...], preferred_element_type=jnp.float32)


def _node_tables(nodes, we_r, we_s):
    return pl.pallas_call(
        _tables_body,
        grid=(B,),
        in_specs=[
            pl.BlockSpec((1, N_NODES, D_NODE), lambda b: (b, 0, 0)),
            pl.BlockSpec((D_NODE, EDGE_OUT), lambda b: (0, 0)),
            pl.BlockSpec((D_NODE, EDGE_OUT), lambda b: (0, 0)),
        ],
        out_specs=[
            pl.BlockSpec((1, N_NODES, EDGE_OUT), lambda b: (b, 0, 0)),
            pl.BlockSpec((1, N_NODES, EDGE_OUT), lambda b: (b, 0, 0)),
        ],
        out_shape=[
            jax.ShapeDtypeStruct((B, N_NODES, EDGE_OUT), jnp.float32),
            jax.ShapeDtypeStruct((B, N_NODES, EDGE_OUT), jnp.float32),
        ],
    )(nodes, we_r, we_s)


EDGE_BLK = 16000
N_EDGE_BLKS = N_EDGES // EDGE_BLK


def _edge_base_body(edges_t_ref, we_e_t_ref, g_ref, we_g_t_ref, be_ref, out_ref):
    base = jnp.dot(we_e_t_ref[...], edges_t_ref[0],
                   preferred_element_type=jnp.float32)
    gterm = jnp.dot(we_g_t_ref[...], g_ref[0],
                    preferred_element_type=jnp.float32)
    out_ref[0] = base + gterm + be_ref[...]


def _edge_base(edges_t, we_e_t, gcol, we_g_t, be_col):
    return pl.pallas_call(
        _edge_base_body,
        grid=(B, N_EDGE_BLKS),
        in_specs=[
            pl.BlockSpec((1, D_EDGE, EDGE_BLK), lambda b, i: (b, 0, i)),
            pl.BlockSpec((EDGE_OUT, D_EDGE), lambda b, i: (0, 0)),
            pl.BlockSpec((1, D_GLOBAL, 1), lambda b, i: (b, 0, 0)),
            pl.BlockSpec((EDGE_OUT, D_GLOBAL), lambda b, i: (0, 0)),
            pl.BlockSpec((EDGE_OUT, 1), lambda b, i: (0, 0)),
        ],
        out_specs=pl.BlockSpec((1, EDGE_OUT, EDGE_BLK), lambda b, i: (b, 0, i)),
        out_shape=jax.ShapeDtypeStruct((B, EDGE_OUT, N_EDGES), jnp.float32),
    )(edges_t, we_e_t, gcol, we_g_t, be_col)


def _sc_edges(base_pk, tabr_flat, tabs_flat, grecv2, gsend2, lrecv2):
    mesh = plsc.VectorSubcoreMesh(core_axis_name="c", subcore_axis_name="s")

    @functools.partial(
        pl.kernel,
        out_type=[
            jax.ShapeDtypeStruct((B, 2, N_ETILES, 8, ET), jnp.float32),
            jax.ShapeDtypeStruct((B * N_NODES, EDGE_OUT), jnp.float32),
        ],
        mesh=mesh,
        scratch_types=[
            pltpu.VMEM((3, K_SUB, ET), jnp.int32),
            pltpu.VMEM((3, K_SUB, ET), jnp.int32),
            pltpu.VMEM((3, K_SUB, ET), jnp.int32),
            pltpu.VMEM((3, 2, K_SUB, 8, ET + 9), jnp.float32),
            pltpu.VMEM((2, CHUNK, EDGE_OUT), jnp.float32),
            pltpu.VMEM((2, CHUNK, EDGE_OUT), jnp.float32),
            pltpu.VMEM_SHARED((N_NODES, EDGE_OUT), jnp.float32),
            pltpu.SemaphoreType.DMA,
            pltpu.SemaphoreType.DMA,
            pltpu.SemaphoreType.DMA,
            pltpu.SemaphoreType.DMA,
            pltpu.SemaphoreType.DMA,
            pltpu.SemaphoreType.DMA,
        ],
        compiler_params=pltpu.CompilerParams(use_tc_tiling_on_sc=False,
                                             needs_layout_passes=False),
    )
    def sc_kernel(base_hbm, tabr_hbm, tabs_hbm, grecv_hbm, gsend_hbm,
                  lrecv_hbm, eout_hbm, agg_hbm, idxr_v, idxs_v, idxl_v,
                  bh, rbuf, sbuf, agg_sh, sem_pf0, sem_pf1, sem_pf2,
                  sem_g, sem_o, sem_sc):
        c = lax.axis_index("c")
        s = lax.axis_index("s")
        sem_pf = [sem_pf0, sem_pf1, sem_pf2]

        npt = N_NODES // NUM_TILES

        def _zero(j, _):
            rbuf[0, j, :] = jnp.zeros((EDGE_OUT,), jnp.float32)
            return 0
        lax.fori_loop(0, npt, _zero, 0)
        pltpu.sync_copy(rbuf.at[0, pl.ds(0, npt)],
                        agg_sh.at[pl.ds(s * npt, npt)])
        plsc.subcore_barrier()

        iota = lax.iota(jnp.int32, 16)
        hvec = lax.shift_right_logical(iota, 3)
        rvec = lax.bitwise_and(iota, 7)

        def et0_of(k):
            return pl.multiple_of(s * ET_PER_TILE + k * K_SUB, K_SUB)

        def fire_pf(k):
            p = k % 3
            et0 = et0_of(k)
            row0 = c * N_ETILES + et0
            ds = [
                pltpu.async_copy(grecv_hbm.at[pl.ds(row0, K_SUB)],
                                 idxr_v.at[p], sem_pf[p]),
                pltpu.async_copy(gsend_hbm.at[pl.ds(row0, K_SUB)],
                                 idxs_v.at[p], sem_pf[p]),
                pltpu.async_copy(lrecv_hbm.at[pl.ds(row0, K_SUB)],
                                 idxl_v.at[p], sem_pf[p]),
            ]
            for h in range(2):
                ds.append(pltpu.async_copy(
                    base_hbm.at[c, h, pl.ds(et0, K_SUB)],
                    bh.at[p, h, :, :, pl.ds(0, ET)], sem_pf[p]))
            return ds

        def fire_g(k):
            p, q = k % 3, k % 2
            ds = []
            for j in range(K_SUB):
                ds.append(pltpu.async_copy(tabr_hbm.at[idxr_v.at[p, j]],
                                           rbuf.at[q, pl.ds(j * ET, ET)],
                                           sem_g))
                ds.append(pltpu.async_copy(tabs_hbm.at[idxs_v.at[p, j]],
                                           sbuf.at[q, pl.ds(j * ET, ET)],
                                           sem_g))
            return ds

        def compute(k):
            p, q = k % 3, k % 2
            bh_p = bh.at[p]

            def et_body(t, _):
                tvec = jnp.full((16,), t, jnp.int32)
                j0 = t * ET

                @plsc.parallel_loop(0, ET, unroll=8)
                def _(l):
                    lvec = jnp.full((16,), l, jnp.int32)
                    base_v = plsc.load_gather(bh_p, [hvec, tvec, rvec, lvec])
                    j = j0 + l
                    v = jnp.maximum(base_v + rbuf[q, j, :] + sbuf[q, j, :], 0.0)
                    plsc.store_scatter(bh_p, [hvec, tvec, rvec, lvec], v)
                    rbuf[q, j, :] = v
                return 0
            lax.fori_loop(0, K_SUB, et_body, 0)

        def fire_o(k):
            p, q = k % 3, k % 2
            et0 = et0_of(k)
            ds = []
            for h in range(2):
                ds.append(pltpu.async_copy(bh.at[p, h, :, :, pl.ds(0, ET)],
                                           eout_hbm.at[c, h, pl.ds(et0, K_SUB)],
                                           sem_o))
            for j in range(K_SUB):
                ds.append(pltpu.async_copy(rbuf.at[q, pl.ds(j * ET, ET)],
                                           agg_sh.at[idxl_v.at[p, j]],
                                           sem_sc, add=True))
            return ds

        def drain(ds):
            for d in ds:
                d.wait()

        pf_d = {0: fire_pf(0), 1: fire_pf(1)}
        drain(pf_d.pop(0))
        g_d = {0: fire_g(0)}
        o_prev = None
        for k in range(N_CHUNKS):
            drain(g_d.pop(k))
            compute(k)
            if o_prev is not None:
                drain(o_prev)
            o_prev = fire_o(k)
            if k + 2 < N_CHUNKS:
                pf_d[k + 2] = fire_pf(k + 2)
            if k + 1 < N_CHUNKS:
                drain(pf_d.pop(k + 1))
                g_d[k + 1] = fire_g(k + 1)
        drain(o_prev)

        @pl.when(s < REM_ETILES)
        def _():
            et0 = NUM_TILES * ET_PER_TILE + s
            row0 = c * N_ETILES + et0
            pltpu.sync_copy(grecv_hbm.at[pl.ds(row0, 1)],
                            idxr_v.at[0, pl.ds(0, 1)])
            pltpu.sync_copy(gsend_hbm.at[pl.ds(row0, 1)],
                            idxs_v.at[0, pl.ds(0, 1)])
            pltpu.sync_copy(lrecv_hbm.at[pl.ds(row0, 1)],
                            idxl_v.at[0, pl.ds(0, 1)])
            for h in range(2):
                pltpu.sync_copy(base_hbm.at[c, h, pl.ds(et0, 1)],
                                bh.at[0, h, pl.ds(0, 1), :, pl.ds(0, ET)])
            drain([pltpu.async_copy(tabr_hbm.at[idxr_v.at[0, 0]],
                                    rbuf.at[0, pl.ds(0, ET)], sem_g),
                   pltpu.async_copy(tabs_hbm.at[idxs_v.at[0, 0]],
                                    sbuf.at[0, pl.ds(0, ET)], sem_g)])
            bh_p = bh.at[0]

            def row_body(j, _):
                lvec = jnp.full((16,), j, jnp.int32)
                tvec = jnp.zeros((16,), jnp.int32)
                base_v = plsc.load_gather(bh_p, [hvec, tvec, rvec, lvec])
                v = jnp.maximum(base_v + rbuf[0, j, :] + sbuf[0, j, :], 0.0)
                plsc.store_scatter(bh_p, [hvec, tvec, rvec, lvec], v)
                rbuf[0, j, :] = v
                return 0
            lax.fori_loop(0, ET, row_body, 0)
            for h in range(2):
                pltpu.sync_copy(bh.at[0, h, pl.ds(0, 1), :, pl.ds(0, ET)],
                                eout_hbm.at[c, h, pl.ds(et0, 1)])
            pltpu.sync_copy(rbuf.at[0, pl.ds(0, ET)],
                            agg_sh.at[idxl_v.at[0, 0]], add=True)

        plsc.subcore_barrier()
        n0 = s * npt
        pltpu.sync_copy(agg_sh.at[pl.ds(n0, npt)],
                        agg_hbm.at[pl.ds(c * N_NODES + n0, npt)])

    return sc_kernel(base_pk, tabr_flat, tabs_flat, grecv2, gsend2, lrecv2)


NODE_BLK = 1000
N_NODE_BLKS = N_NODES // NODE_BLK


def _node_global_body(agg_ref, nodes_ref, g_ref, wn_a_ref, wn_n_ref, wn_g_ref,
                      bn_ref, wg_e_ref, wg_n_ref, wg_g_ref, bg_ref,
                      nout_ref, gout_ref, nsum_ref, esum_ref):
    b = pl.program_id(0)
    i = pl.program_id(1)
    agg = agg_ref[0]
    x = (jnp.dot(agg, wn_a_ref[...], preferred_element_type=jnp.float32)
         + jnp.dot(nodes_ref[0], wn_n_ref[...], preferred_element_type=jnp.float32)
         + jnp.dot(g_ref[0], wn_g_ref[...], preferred_element_type=jnp.float32)
         + bn_ref[...])
    n_out = jnp.maximum(x, 0.0)
    nout_ref[0] = n_out
    ns = jnp.sum(n_out, axis=0, keepdims=True)
    es = jnp.sum(agg, axis=0, keepdims=True)

    @pl.when(i == 0)
    def _():
        nsum_ref[...] = ns
        esum_ref[...] = es

    @pl.when(i > 0)
    def _():
        nsum_ref[...] += ns
        esum_ref[...] += es

    @pl.when(i == N_NODE_BLKS - 1)
    def _():
        gi = (jnp.dot(esum_ref[...], wg_e_ref[...], preferred_element_type=jnp.float32)
              + jnp.dot(nsum_ref[...], wg_n_ref[...], preferred_element_type=jnp.float32)
              + jnp.dot(g_ref[0], wg_g_ref[...], preferred_element_type=jnp.float32)
              + bg_ref[...])
        gout_ref[pl.ds(b, 1), :] = jnp.maximum(gi, 0.0)


def _node_global(agg, nodes, g3, wn_a, wn_n, wn_g, bn2, wg_e, wg_n, wg_g, bg2):
    return pl.pallas_call(
        _node_global_body,
        grid=(B, N_NODE_BLKS),
        in_specs=[
            pl.BlockSpec((1, NODE_BLK, EDGE_OUT), lambda b, i: (b, i, 0)),
            pl.BlockSpec((1, NODE_BLK, D_NODE), lambda b, i: (b, i, 0)),
            pl.BlockSpec((1, 1, D_GLOBAL), lambda b, i: (b, 0, 0)),
            pl.BlockSpec((EDGE_OUT, NODE_OUT), lambda b, i: (0, 0)),
            pl.BlockSpec((D_NODE, NODE_OUT), lambda b, i: (0, 0)),
            pl.BlockSpec((D_GLOBAL, NODE_OUT), lambda b, i: (0, 0)),
            pl.BlockSpec((1, NODE_OUT), lambda b, i: (0, 0)),
            pl.BlockSpec((EDGE_OUT, GLOBAL_OUT), lambda b, i: (0, 0)),
            pl.BlockSpec((NODE_OUT, GLOBAL_OUT), lambda b, i: (0, 0)),
            pl.BlockSpec((D_GLOBAL, GLOBAL_OUT), lambda b, i: (0, 0)),
            pl.BlockSpec((1, GLOBAL_OUT), lambda b, i: (0, 0)),
        ],
        out_specs=[
            pl.BlockSpec((1, NODE_BLK, NODE_OUT), lambda b, i: (b, i, 0)),
            pl.BlockSpec((B, GLOBAL_OUT), lambda b, i: (0, 0)),
        ],
        out_shape=[
            jax.ShapeDtypeStruct((B, N_NODES, NODE_OUT), jnp.float32),
            jax.ShapeDtypeStruct((B, GLOBAL_OUT), jnp.float32),
        ],
        scratch_shapes=[
            pltpu.VMEM((1, NODE_OUT), jnp.float32),
            pltpu.VMEM((1, EDGE_OUT), jnp.float32),
        ],
    )(agg, nodes, g3, wn_a, wn_n, wn_g, bn2, wg_e, wg_n, wg_g, bg2)


def kernel(nodes, edges, senders, receivers, global_feats, We, be, Wn, bn, Wg, bg):
    we_e, we_r = We[0:D_EDGE], We[D_EDGE:D_EDGE + D_NODE]
    we_s = We[D_EDGE + D_NODE:D_EDGE + 2 * D_NODE]
    we_g = We[D_EDGE + 2 * D_NODE:]
    wn_a, wn_n, wn_g = Wn[0:EDGE_OUT], Wn[EDGE_OUT:EDGE_OUT + D_NODE], Wn[EDGE_OUT + D_NODE:]
    wg_e, wg_n, wg_g = Wg[0:EDGE_OUT], Wg[EDGE_OUT:EDGE_OUT + NODE_OUT], Wg[EDGE_OUT + NODE_OUT:]

    g3 = global_feats.reshape(B, 1, D_GLOBAL)
    gcol = global_feats.reshape(B, D_GLOBAL, 1)
    be_col = be.reshape(EDGE_OUT, 1)
    bn2 = bn.reshape(1, NODE_OUT)
    bg2 = bg.reshape(1, GLOBAL_OUT)

    tabr, tabs = _node_tables(nodes, we_r, we_s)
    base_t = _edge_base(jnp.swapaxes(edges, 1, 2), we_e.T, gcol, we_g.T, be_col)
    base_pk = base_t.reshape(B, 2, 8, N_ETILES, ET).transpose(0, 1, 3, 2, 4)

    b_off = (jnp.arange(B, dtype=jnp.int32) * N_NODES)[:, None]
    grecv2 = (receivers + b_off).reshape(B * N_EDGES // ET, ET)
    gsend2 = (senders + b_off).reshape(B * N_EDGES // ET, ET)
    lrecv2 = receivers.reshape(B * N_EDGES // ET, ET)

    eout_pk, agg_flat = _sc_edges(
        base_pk,
        tabr.reshape(B * N_NODES, EDGE_OUT),
        tabs.reshape(B * N_NODES, EDGE_OUT),
        grecv2, gsend2, lrecv2)

    agg = agg_flat.reshape(B, N_NODES, EDGE_OUT)
    e_out = eout_pk.transpose(0, 2, 4, 1, 3).reshape(B, N_EDGES, EDGE_OUT)

    n_out, g_out = _node_global(agg, nodes, g3, wn_a, wn_n, wn_g, bn2,
                                wg_e, wg_n, wg_g, bg2)
    return (n_out, e_out, g_out)
